# baseline (speedup 1.0000x reference)
import jax
import jax.numpy as jnp
from jax.experimental import pallas as pl
from jax.experimental.pallas import tpu as pltpu

IN_F = 4096
OUT_F = 4096
NB = 8
BLK = 512
TOPK = 8
ALPHA = 2.0
NT = 16384

XR = NT // NB


def _prep_kernel(wp_ref, wv_ref, wb_ref, x_ref, w_out_ref, x_out_ref):
    o = pl.program_id(0)
    i = pl.program_id(1)
    j = o * NB + i
    wp = wp_ref[...]
    col = jax.lax.broadcasted_iota(jnp.int32, wp.shape, 1)
    mx = jnp.max(wp, axis=1, keepdims=True)
    idx = jnp.min(jnp.where(wp == mx, col, wp.shape[1]), axis=1, keepdims=True)
    onehot = (col == idx).astype(wp.dtype)
    mag_row = jnp.sum(wp * (onehot * (ALPHA - 1.0) + 1.0), axis=0,
                      keepdims=True)
    mag_j = jnp.sum(jnp.where(col[:1] == j, mag_row, 0.0))
    sel = jnp.sum(jnp.where(col == j, onehot, 0.0), axis=1,
                  keepdims=True)
    delta = jnp.sum(sel[:, :, None] * wv_ref[...], axis=0)
    w_out_ref[...] = (wb_ref[...] + mag_j * delta).astype(jnp.bfloat16)
    x_out_ref[...] = x_ref[...].astype(jnp.bfloat16)


def _matmul_kernel(x_ref, w_ref, b_ref, out_ref):
    acc = jax.lax.dot_general(
        x_ref[...], w_ref[...], (((1,), (1,)), ((), ())),
        preferred_element_type=jnp.float32)
    out_ref[...] = acc + b_ref[...]


def kernel(x, W_base, b_base, bola_w_p, bola_w_v):
    w_eff, xb = pl.pallas_call(
        _prep_kernel,
        grid=(NB, NB),
        in_specs=[
            pl.BlockSpec((TOPK, NB * NB), lambda o, i: (0, 0)),
            pl.BlockSpec((TOPK, BLK, BLK), lambda o, i: (0, 0, 0)),
            pl.BlockSpec((BLK, BLK), lambda o, i: (o, i)),
            pl.BlockSpec((XR, BLK), lambda o, i: (o, i)),
        ],
        out_specs=[
            pl.BlockSpec((BLK, BLK), lambda o, i: (o, i)),
            pl.BlockSpec((XR, BLK), lambda o, i: (o, i)),
        ],
        out_shape=[
            jax.ShapeDtypeStruct((OUT_F, IN_F), jnp.bfloat16),
            jax.ShapeDtypeStruct((NT, IN_F), jnp.bfloat16),
        ],
    )(bola_w_p, bola_w_v, W_base, x)

    b2 = b_base.reshape(1, OUT_F)
    bm, bn = 2048, 512
    out = pl.pallas_call(
        _matmul_kernel,
        grid=(NT // bm, OUT_F // bn),
        in_specs=[
            pl.BlockSpec((bm, IN_F), lambda m, n: (m, 0)),
            pl.BlockSpec((bn, IN_F), lambda m, n: (n, 0)),
            pl.BlockSpec((1, bn), lambda m, n: (0, n)),
        ],
        out_specs=pl.BlockSpec((bm, bn), lambda m, n: (m, n)),
        out_shape=jax.ShapeDtypeStruct((NT, OUT_F), jnp.float32),
        compiler_params=pltpu.CompilerParams(
            dimension_semantics=("parallel", "parallel")),
    )(xb, w_eff, b2)
    return out


# speedup vs baseline: 1.0177x; 1.0177x over previous
import jax
import jax.numpy as jnp
from jax.experimental import pallas as pl
from jax.experimental.pallas import tpu as pltpu

IN_F = 4096
OUT_F = 4096
NB = 8
BLK = 512
TOPK = 8
ALPHA = 2.0
NT = 16384


def _assemble_kernel(wp_ref, wv_ref, wb_ref, out_ref):
    o = pl.program_id(0)
    i = pl.program_id(1)
    j = o * NB + i
    wp = wp_ref[...]
    col = jax.lax.broadcasted_iota(jnp.int32, wp.shape, 1)
    mx = jnp.max(wp, axis=1, keepdims=True)
    idx = jnp.min(jnp.where(wp == mx, col, wp.shape[1]), axis=1, keepdims=True)
    onehot = (col == idx).astype(wp.dtype)
    mag_row = jnp.sum(wp * (onehot * (ALPHA - 1.0) + 1.0), axis=0,
                      keepdims=True)
    mag_j = jnp.sum(jnp.where(col[:1] == j, mag_row, 0.0))
    sel = jnp.sum(jnp.where(col == j, onehot, 0.0), axis=1,
                  keepdims=True)
    any_sel = jnp.sum(sel) > 0.0

    @pl.when(any_sel)
    def _():
        delta = jnp.sum(sel[:, :, None] * wv_ref[...], axis=0)
        out_ref[...] = (wb_ref[...] + mag_j * delta).astype(jnp.bfloat16)

    @pl.when(jnp.logical_not(any_sel))
    def _():
        out_ref[...] = wb_ref[...].astype(jnp.bfloat16)


def _matmul_kernel(x_ref, w_ref, b_ref, out_ref, xb_ref):
    n = pl.program_id(1)

    @pl.when(n == 0)
    def _():
        xb_ref[...] = x_ref[...].astype(jnp.bfloat16)

    acc = jax.lax.dot_general(
        xb_ref[...], w_ref[...], (((1,), (1,)), ((), ())),
        preferred_element_type=jnp.float32)
    out_ref[...] = acc + b_ref[...]


def kernel(x, W_base, b_base, bola_w_p, bola_w_v):
    w_eff = pl.pallas_call(
        _assemble_kernel,
        grid=(NB, NB),
        in_specs=[
            pl.BlockSpec((TOPK, NB * NB), lambda o, i: (0, 0)),
            pl.BlockSpec((TOPK, BLK, BLK), lambda o, i: (0, 0, 0)),
            pl.BlockSpec((BLK, BLK), lambda o, i: (o, i)),
        ],
        out_specs=pl.BlockSpec((BLK, BLK), lambda o, i: (o, i)),
        out_shape=jax.ShapeDtypeStruct((OUT_F, IN_F), jnp.bfloat16),
    )(bola_w_p, bola_w_v, W_base)

    b2 = b_base.reshape(1, OUT_F)
    bm, bn = 1024, 512
    out = pl.pallas_call(
        _matmul_kernel,
        grid=(NT // bm, OUT_F // bn),
        in_specs=[
            pl.BlockSpec((bm, IN_F), lambda m, n: (m, 0)),
            pl.BlockSpec((bn, IN_F), lambda m, n: (n, 0)),
            pl.BlockSpec((1, bn), lambda m, n: (0, n)),
        ],
        out_specs=pl.BlockSpec((bm, bn), lambda m, n: (m, n)),
        out_shape=jax.ShapeDtypeStruct((NT, OUT_F), jnp.float32),
        scratch_shapes=[pltpu.VMEM((bm, IN_F), jnp.bfloat16)],
        compiler_params=pltpu.CompilerParams(
            dimension_semantics=("parallel", "parallel")),
    )(x, w_eff, b2)
    return out


# speedup vs baseline: 1.0939x; 1.0749x over previous
import jax
import jax.numpy as jnp
from jax.experimental import pallas as pl
from jax.experimental.pallas import tpu as pltpu

IN_F = 4096
OUT_F = 4096
NB = 8
BLK = 512
TOPK = 8
ALPHA = 2.0
NT = 16384

BM = 2048
BN = 512
CH = 256
NCH = BM // CH
MSTEPS = NT // BM


def _assemble_kernel(wp_ref, wv_ref, wb_ref, out_ref):
    o = pl.program_id(0)
    i = pl.program_id(1)
    j = o * NB + i
    wp = wp_ref[...]
    col = jax.lax.broadcasted_iota(jnp.int32, wp.shape, 1)
    mx = jnp.max(wp, axis=1, keepdims=True)
    idx = jnp.min(jnp.where(wp == mx, col, wp.shape[1]), axis=1, keepdims=True)
    onehot = (col == idx).astype(wp.dtype)
    mag_row = jnp.sum(wp * (onehot * (ALPHA - 1.0) + 1.0), axis=0,
                      keepdims=True)
    mag_j = jnp.sum(jnp.where(col[:1] == j, mag_row, 0.0))
    sel = jnp.sum(jnp.where(col == j, onehot, 0.0), axis=1,
                  keepdims=True)
    any_sel = jnp.sum(sel) > 0.0

    @pl.when(any_sel)
    def _():
        delta = jnp.sum(sel[:, :, None] * wv_ref[...], axis=0)
        out_ref[...] = (wb_ref[...] + mag_j * delta).astype(jnp.bfloat16)

    @pl.when(jnp.logical_not(any_sel))
    def _():
        out_ref[...] = wb_ref[...].astype(jnp.bfloat16)


def _matmul_kernel(x_hbm, w_ref, b_ref, out_ref, xb0, xb1, xf, sem):
    m = pl.program_id(0)
    n = pl.program_id(1)

    @pl.when(jnp.logical_and(m == 0, n == 0))
    def _():
        def prime(c, carry):
            cpy = pltpu.make_async_copy(
                x_hbm.at[pl.ds(c * CH, CH)], xf.at[0], sem.at[0])
            cpy.start()
            cpy.wait()
            xb0[pl.ds(c * CH, CH), :] = xf[0].astype(jnp.bfloat16)
            return carry
        jax.lax.fori_loop(0, NCH, prime, 0)

    @pl.when(jnp.logical_and(m >= 1, n == 0))
    def _():
        src = x_hbm.at[pl.ds(m * BM + (NCH - 1) * CH, CH)]
        pltpu.make_async_copy(src, xf.at[(NCH - 1) % 2],
                              sem.at[(NCH - 1) % 2]).wait()
        chunk = xf[(NCH - 1) % 2].astype(jnp.bfloat16)

        @pl.when(m % 2 == 0)
        def _():
            xb0[pl.ds((NCH - 1) * CH, CH), :] = chunk

        @pl.when(m % 2 == 1)
        def _():
            xb1[pl.ds((NCH - 1) * CH, CH), :] = chunk

    nxt_valid = m < MSTEPS - 1

    @pl.when(nxt_valid)
    def _():
        src = x_hbm.at[pl.ds((m + 1) * BM + n * CH, CH)]
        pltpu.make_async_copy(src, xf.at[n % 2], sem.at[n % 2]).start()

    @pl.when(jnp.logical_and(nxt_valid, n >= 1))
    def _():
        src = x_hbm.at[pl.ds((m + 1) * BM + (n - 1) * CH, CH)]
        pltpu.make_async_copy(src, xf.at[(n - 1) % 2],
                              sem.at[(n - 1) % 2]).wait()
        chunk = xf[(n - 1) % 2].astype(jnp.bfloat16)

        @pl.when(m % 2 == 0)
        def _():
            xb1[pl.ds((n - 1) * CH, CH), :] = chunk

        @pl.when(m % 2 == 1)
        def _():
            xb0[pl.ds((n - 1) * CH, CH), :] = chunk

    dn = (((1,), (1,)), ((), ()))

    @pl.when(m % 2 == 0)
    def _():
        acc = jax.lax.dot_general(xb0[...], w_ref[...], dn,
                                  preferred_element_type=jnp.float32)
        out_ref[...] = acc + b_ref[...]

    @pl.when(m % 2 == 1)
    def _():
        acc = jax.lax.dot_general(xb1[...], w_ref[...], dn,
                                  preferred_element_type=jnp.float32)
        out_ref[...] = acc + b_ref[...]


def kernel(x, W_base, b_base, bola_w_p, bola_w_v):
    w_eff = pl.pallas_call(
        _assemble_kernel,
        grid=(NB, NB),
        in_specs=[
            pl.BlockSpec((TOPK, NB * NB), lambda o, i: (0, 0)),
            pl.BlockSpec((TOPK, BLK, BLK), lambda o, i: (0, 0, 0)),
            pl.BlockSpec((BLK, BLK), lambda o, i: (o, i)),
        ],
        out_specs=pl.BlockSpec((BLK, BLK), lambda o, i: (o, i)),
        out_shape=jax.ShapeDtypeStruct((OUT_F, IN_F), jnp.bfloat16),
    )(bola_w_p, bola_w_v, W_base)

    b2 = b_base.reshape(1, OUT_F)
    out = pl.pallas_call(
        _matmul_kernel,
        grid=(MSTEPS, OUT_F // BN),
        in_specs=[
            pl.BlockSpec(memory_space=pl.ANY),
            pl.BlockSpec((BN, IN_F), lambda m, n: (n, 0)),
            pl.BlockSpec((1, BN), lambda m, n: (0, n)),
        ],
        out_specs=pl.BlockSpec((BM, BN), lambda m, n: (m, n)),
        out_shape=jax.ShapeDtypeStruct((NT, OUT_F), jnp.float32),
        scratch_shapes=[
            pltpu.VMEM((BM, IN_F), jnp.bfloat16),
            pltpu.VMEM((BM, IN_F), jnp.bfloat16),
            pltpu.VMEM((2, CH, IN_F), jnp.float32),
            pltpu.SemaphoreType.DMA((2,)),
        ],
        compiler_params=pltpu.CompilerParams(
            dimension_semantics=("arbitrary", "arbitrary"),
            vmem_limit_bytes=60 * 1024 * 1024),
    )(x, w_eff, b2)
    return out


# speedup vs baseline: 1.1496x; 1.0509x over previous
import jax
import jax.numpy as jnp
from jax.experimental import pallas as pl
from jax.experimental.pallas import tpu as pltpu

IN_F = 4096
OUT_F = 4096
NB = 8
BLK = 512
TOPK = 8
ALPHA = 2.0
NT = 16384

BM = 2048
BN = 512
CH = 256
NCH = BM // CH
MSTEPS = NT // BM


def _assemble_kernel(wp_ref, wv_ref, wb_ref, out_ref):
    o = pl.program_id(0)
    wp = wp_ref[...]
    col = jax.lax.broadcasted_iota(jnp.int32, wp.shape, 1)
    mx = jnp.max(wp, axis=1, keepdims=True)
    idx = jnp.min(jnp.where(wp == mx, col, wp.shape[1]), axis=1, keepdims=True)
    onehot = (col == idx).astype(wp.dtype)
    mag_row = jnp.sum(wp * (onehot * (ALPHA - 1.0) + 1.0), axis=0,
                      keepdims=True)
    for i in range(NB):
        j = o * NB + i
        mag_j = jnp.sum(jnp.where(col[:1] == j, mag_row, 0.0))
        sel = jnp.sum(jnp.where(col == j, onehot, 0.0), axis=1,
                      keepdims=True)
        any_sel = jnp.sum(sel) > 0.0
        cs = slice(i * BLK, (i + 1) * BLK)

        @pl.when(any_sel)
        def _(cs=cs, sel=sel, mag_j=mag_j):
            delta = jnp.sum(sel[:, :, None] * wv_ref[...], axis=0)
            out_ref[:, cs] = (wb_ref[:, cs] + mag_j * delta).astype(
                jnp.bfloat16)

        @pl.when(jnp.logical_not(any_sel))
        def _(cs=cs):
            out_ref[:, cs] = wb_ref[:, cs].astype(jnp.bfloat16)


def _matmul_kernel(x_hbm, w_ref, b_ref, out_ref, xb0, xb1, xf, sem):
    m = pl.program_id(0)
    n = pl.program_id(1)

    @pl.when(jnp.logical_and(m == 0, n == 0))
    def _():
        def prime(c, carry):
            cpy = pltpu.make_async_copy(
                x_hbm.at[pl.ds(c * CH, CH)], xf.at[0], sem.at[0])
            cpy.start()
            cpy.wait()
            xb0[pl.ds(c * CH, CH), :] = xf[0].astype(jnp.bfloat16)
            return carry
        jax.lax.fori_loop(0, NCH, prime, 0)

    @pl.when(jnp.logical_and(m >= 1, n == 0))
    def _():
        src = x_hbm.at[pl.ds(m * BM + (NCH - 1) * CH, CH)]
        pltpu.make_async_copy(src, xf.at[(NCH - 1) % 2],
                              sem.at[(NCH - 1) % 2]).wait()
        chunk = xf[(NCH - 1) % 2].astype(jnp.bfloat16)

        @pl.when(m % 2 == 0)
        def _():
            xb0[pl.ds((NCH - 1) * CH, CH), :] = chunk

        @pl.when(m % 2 == 1)
        def _():
            xb1[pl.ds((NCH - 1) * CH, CH), :] = chunk

    nxt_valid = m < MSTEPS - 1

    @pl.when(nxt_valid)
    def _():
        src = x_hbm.at[pl.ds((m + 1) * BM + n * CH, CH)]
        pltpu.make_async_copy(src, xf.at[n % 2], sem.at[n % 2]).start()

    @pl.when(jnp.logical_and(nxt_valid, n >= 1))
    def _():
        src = x_hbm.at[pl.ds((m + 1) * BM + (n - 1) * CH, CH)]
        pltpu.make_async_copy(src, xf.at[(n - 1) % 2],
                              sem.at[(n - 1) % 2]).wait()
        chunk = xf[(n - 1) % 2].astype(jnp.bfloat16)

        @pl.when(m % 2 == 0)
        def _():
            xb1[pl.ds((n - 1) * CH, CH), :] = chunk

        @pl.when(m % 2 == 1)
        def _():
            xb0[pl.ds((n - 1) * CH, CH), :] = chunk

    dn = (((1,), (1,)), ((), ()))

    @pl.when(m % 2 == 0)
    def _():
        acc = jax.lax.dot_general(xb0[...], w_ref[...], dn,
                                  preferred_element_type=jnp.float32)
        out_ref[...] = acc + b_ref[...]

    @pl.when(m % 2 == 1)
    def _():
        acc = jax.lax.dot_general(xb1[...], w_ref[...], dn,
                                  preferred_element_type=jnp.float32)
        out_ref[...] = acc + b_ref[...]


def kernel(x, W_base, b_base, bola_w_p, bola_w_v):
    w_eff = pl.pallas_call(
        _assemble_kernel,
        grid=(NB,),
        in_specs=[
            pl.BlockSpec((TOPK, NB * NB), lambda o: (0, 0)),
            pl.BlockSpec((TOPK, BLK, BLK), lambda o: (0, 0, 0)),
            pl.BlockSpec((BLK, IN_F), lambda o: (o, 0)),
        ],
        out_specs=pl.BlockSpec((BLK, IN_F), lambda o: (o, 0)),
        out_shape=jax.ShapeDtypeStruct((OUT_F, IN_F), jnp.bfloat16),
    )(bola_w_p, bola_w_v, W_base)

    b2 = b_base.reshape(1, OUT_F)
    out = pl.pallas_call(
        _matmul_kernel,
        grid=(MSTEPS, OUT_F // BN),
        in_specs=[
            pl.BlockSpec(memory_space=pl.ANY),
            pl.BlockSpec((BN, IN_F), lambda m, n: (n, 0)),
            pl.BlockSpec((1, BN), lambda m, n: (0, n)),
        ],
        out_specs=pl.BlockSpec((BM, BN), lambda m, n: (m, n)),
        out_shape=jax.ShapeDtypeStruct((NT, OUT_F), jnp.float32),
        scratch_shapes=[
            pltpu.VMEM((BM, IN_F), jnp.bfloat16),
            pltpu.VMEM((BM, IN_F), jnp.bfloat16),
            pltpu.VMEM((2, CH, IN_F), jnp.float32),
            pltpu.SemaphoreType.DMA((2,)),
        ],
        compiler_params=pltpu.CompilerParams(
            dimension_semantics=("arbitrary", "arbitrary"),
            vmem_limit_bytes=60 * 1024 * 1024),
    )(x, w_eff, b2)
    return out
